# baseline (speedup 1.0000x reference)
import functools

import jax
import jax.numpy as jnp
from jax import lax
from jax.experimental import pallas as pl
from jax.experimental.pallas import tpu as pltpu
from jax.experimental.pallas import tpu_sc as plsc

D_MODEL = 128
NUM_CORES = 2
NUM_SUBCORES = 16
NUM_WORKERS = NUM_CORES * NUM_SUBCORES
CHUNK = 128
NBUF = 5


@functools.cache
def _build(total_rows):
    nchunk_w = total_rows // (NUM_WORKERS * CHUNK)
    rows_w = nchunk_w * CHUNK
    mesh = plsc.VectorSubcoreMesh(core_axis_name="c", subcore_axis_name="s")

    @functools.partial(
        pl.kernel,
        mesh=mesh,
        out_type=jax.ShapeDtypeStruct((total_rows, D_MODEL), jnp.float32),
        scratch_types=[
            pltpu.VMEM((rows_w,), jnp.int32),
            pltpu.VMEM((NBUF, CHUNK, D_MODEL), jnp.float32),
            pltpu.SemaphoreType.DMA((NBUF,)),
            pltpu.SemaphoreType.DMA((NBUF,)),
        ],
    )
    def gather_kernel(idx_hbm, table_hbm, out_hbm, idx_v, rows_v, gsem, ssem):
        wid = lax.axis_index("s") * NUM_CORES + lax.axis_index("c")
        base = wid * rows_w
        pltpu.sync_copy(idx_hbm.at[pl.ds(base, rows_w)], idx_v)

        def g_copy(j, b):
            return pltpu.make_async_copy(
                table_hbm.at[idx_v.at[pl.ds(j * CHUNK, CHUNK)]],
                rows_v.at[b],
                gsem.at[b],
            )

        def s_copy(j, b):
            return pltpu.make_async_copy(
                rows_v.at[b],
                out_hbm.at[pl.ds(base + j * CHUNK, CHUNK)],
                ssem.at[b],
            )

        for b in range(NBUF):
            g_copy(b, b).start()

        def super_step(i, carry):
            for b in range(NBUF):
                j = i * NBUF + b
                bn = (b - 1) % NBUF

                @pl.when(j >= 1)
                def _():
                    s_copy(j - 1, bn).wait()

                @pl.when(jnp.logical_and(j >= 1, j + NBUF - 1 < nchunk_w))
                def _():
                    g_copy(j + NBUF - 1, bn).start()

                g_copy(j, b).wait()
                s_copy(j, b).start()
            return carry

        lax.fori_loop(0, nchunk_w // NBUF, super_step, 0)
        s_copy(nchunk_w - 1, (nchunk_w - 1) % NBUF).wait()

    return gather_kernel


def kernel(token_ids, weight):
    b, s = token_ids.shape
    total = b * s
    idx = token_ids.reshape(total).astype(jnp.int32)
    out = _build(total)(idx, weight)
    return out.reshape(b, s, D_MODEL)


# speedup vs baseline: 1.0132x; 1.0132x over previous
import functools

import jax
import jax.numpy as jnp
from jax import lax
from jax.experimental import pallas as pl
from jax.experimental.pallas import tpu as pltpu
from jax.experimental.pallas import tpu_sc as plsc

D_MODEL = 128
NUM_CORES = 2
NUM_SUBCORES = 16
NUM_WORKERS = NUM_CORES * NUM_SUBCORES
CHUNK = 128
NBUF = 7


@functools.cache
def _build(total_rows):
    nchunk_w = total_rows // (NUM_WORKERS * CHUNK)
    rows_w = nchunk_w * CHUNK
    nsuper = nchunk_w // NBUF
    mesh = plsc.VectorSubcoreMesh(core_axis_name="c", subcore_axis_name="s")

    @functools.partial(
        pl.kernel,
        mesh=mesh,
        out_type=jax.ShapeDtypeStruct((total_rows, D_MODEL), jnp.float32),
        scratch_types=[
            pltpu.VMEM((rows_w,), jnp.int32),
            pltpu.VMEM((NBUF, CHUNK, D_MODEL), jnp.float32),
            pltpu.SemaphoreType.DMA((NBUF,)),
            pltpu.SemaphoreType.DMA((NBUF,)),
        ],
    )
    def gather_kernel(idx_hbm, table_hbm, out_hbm, idx_v, rows_v, gsem, ssem):
        wid = lax.axis_index("s") * NUM_CORES + lax.axis_index("c")
        base = wid * rows_w
        pltpu.sync_copy(idx_hbm.at[pl.ds(base, rows_w)], idx_v)

        def g_copy(j, b):
            return pltpu.make_async_copy(
                table_hbm.at[idx_v.at[pl.ds(j * CHUNK, CHUNK)]],
                rows_v.at[b],
                gsem.at[b],
            )

        def s_copy(j, b):
            return pltpu.make_async_copy(
                rows_v.at[b],
                out_hbm.at[pl.ds(base + j * CHUNK, CHUNK)],
                ssem.at[b],
            )

        for b in range(NBUF):
            g_copy(b, b).start()

        def super_step(i, carry):
            for b in range(NBUF):
                j = i * NBUF + b
                bn = (b - 1) % NBUF

                @pl.when(j >= 1)
                def _():
                    s_copy(j - 1, bn).wait()

                @pl.when(jnp.logical_and(j >= 1, j + NBUF - 1 < nchunk_w))
                def _():
                    g_copy(j + NBUF - 1, bn).start()

                g_copy(j, b).wait()
                s_copy(j, b).start()
            return carry

        lax.fori_loop(0, nsuper, super_step, 0)
        for j in range(nsuper * NBUF, nchunk_w):
            s_copy(j - 1, (j - 1) % NBUF).wait()
            g_copy(j, j % NBUF).wait()
            s_copy(j, j % NBUF).start()
        s_copy(nchunk_w - 1, (nchunk_w - 1) % NBUF).wait()

    return gather_kernel


def kernel(token_ids, weight):
    b, s = token_ids.shape
    total = b * s
    idx = token_ids.reshape(total).astype(jnp.int32)
    out = _build(total)(idx, weight)
    return out.reshape(b, s, D_MODEL)
